# Initial kernel scaffold; baseline (speedup 1.0000x reference)
import jax
import jax.numpy as jnp
from jax.experimental import pallas as pl


def kernel(x, edge_index_r0, edge_index_r1, edge_index_r2, W0_r0, b0_r0, W0_r1, b0_r1, W0_r2, b0_r2, W1_r0, b1_r0, W1_r1, b1_r1, W1_r2, b1_r2):
    raise NotImplementedError("write your pallas kernel here")


# speedup vs baseline: 1.5615x; 1.5615x over previous
import functools

import jax
import jax.numpy as jnp
from jax import lax
from jax.experimental import pallas as pl
from jax.experimental.pallas import tpu as pltpu
from jax.experimental.pallas import tpu_sc as plsc

N = 10000
D = 256
HW = 128
E = 64000

NC = 2
NS = 16
HALF = N // NC
ACC_ROWS = 5120
GARBAGE = HALF
ZROWS = ACC_ROWS // NS
EPT = E // NS
K = 80
NCHUNK = EPT // K
OUT_T = 5
OUT_ROWS = HALF // OUT_T


def _sc_segsum(compute_deg):
    mesh = plsc.VectorSubcoreMesh(core_axis_name="c", subcore_axis_name="s")
    out_type = [jax.ShapeDtypeStruct((N, D), jnp.float32) for _ in range(3)]
    if compute_deg:
        out_type += [jax.ShapeDtypeStruct((N, HW), jnp.float32) for _ in range(3)]
    scratch = [
        pltpu.VMEM((K,), jnp.int32),
        pltpu.VMEM((K,), jnp.int32),
        pltpu.VMEM((K,), jnp.int32),
        pltpu.VMEM((K, HW), jnp.float32),
        pltpu.VMEM((K, HW), jnp.float32),
        pltpu.VMEM((K, HW), jnp.float32),
        pltpu.VMEM_SHARED((ACC_ROWS, HW), jnp.float32),
        pltpu.VMEM_SHARED((ACC_ROWS, HW), jnp.float32),
        pltpu.SemaphoreType.DMA,
    ]

    @functools.partial(pl.kernel, mesh=mesh, out_type=out_type,
                       scratch_types=scratch)
    def k(feat_h, s0, d0, s1, d1, s2, d2, z_h, o_h, *rest):
        aggs = rest[0:3]
        degs = rest[3:6] if compute_deg else (None, None, None)
        srcb, dstb, ldb, rlo, rhi, onesb, alo, ahi, sem = rest[6 if compute_deg else 3:]

        c = lax.axis_index("c")
        s = lax.axis_index("s")
        lo = c * HALF

        def route(i):
            for g in range(K // 16):
                dvec = dstb[pl.ds(g * 16, 16)]
                m = (dvec >= lo) & (dvec < lo + HALF)
                ldb[pl.ds(g * 16, 16)] = jnp.where(m, dvec - lo, GARBAGE)

        if compute_deg:
            pltpu.sync_copy(o_h, onesb)
            for (dsth, degh) in zip((d0, d1, d2), degs):
                pltpu.sync_copy(z_h, alo.at[pl.ds(s * ZROWS, ZROWS)])
                plsc.subcore_barrier()

                def dbody(i, carry):
                    base = pl.multiple_of(s * EPT + i * K, 8)
                    pltpu.sync_copy(dsth.at[pl.ds(base, K)], dstb)
                    route(i)
                    pltpu.sync_copy(onesb, alo.at[ldb], add=True)
                    return carry

                lax.fori_loop(0, NCHUNK, dbody, 0)
                plsc.subcore_barrier()

                @pl.when(s < OUT_T)
                def _():
                    r0 = pl.multiple_of(s * OUT_ROWS, 8)
                    pltpu.sync_copy(alo.at[pl.ds(r0, OUT_ROWS)],
                                    degh.at[pl.ds(lo + r0, OUT_ROWS)])

                plsc.subcore_barrier()

        for (srch, dsth, aggh) in zip((s0, s1, s2), (d0, d1, d2), aggs):
            pltpu.sync_copy(z_h, alo.at[pl.ds(s * ZROWS, ZROWS)])
            pltpu.sync_copy(z_h, ahi.at[pl.ds(s * ZROWS, ZROWS)])
            plsc.subcore_barrier()

            def body(i, carry):
                base = pl.multiple_of(s * EPT + i * K, 8)
                pltpu.sync_copy(srch.at[pl.ds(base, K)], srcb)
                pltpu.sync_copy(dsth.at[pl.ds(base, K)], dstb)
                pltpu.async_copy(feat_h.at[srcb, pl.ds(0, HW)], rlo, sem).wait()
                pltpu.async_copy(feat_h.at[srcb, pl.ds(HW, HW)], rhi, sem).wait()
                route(i)
                pltpu.sync_copy(rlo, alo.at[ldb], add=True)
                pltpu.sync_copy(rhi, ahi.at[ldb], add=True)
                return carry

            lax.fori_loop(0, NCHUNK, body, 0)
            plsc.subcore_barrier()

            @pl.when(s < OUT_T)
            def _():
                r0 = pl.multiple_of(s * OUT_ROWS, 8)
                pltpu.sync_copy(alo.at[pl.ds(r0, OUT_ROWS)],
                                aggh.at[pl.ds(lo + r0, OUT_ROWS), pl.ds(0, HW)])
                pltpu.sync_copy(ahi.at[pl.ds(r0, OUT_ROWS)],
                                aggh.at[pl.ds(lo + r0, OUT_ROWS), pl.ds(HW, HW)])

            plsc.subcore_barrier()

    return k


_sc_l0 = _sc_segsum(compute_deg=True)
_sc_l1 = _sc_segsum(compute_deg=False)


BLK = 512
_GRID = (N + BLK - 1) // BLK


def _tc_body(a0, a1, a2, g0, g1, g2, w0, w1, w2, b0, b1, b2, o):
    inv0 = 1.0 / jnp.maximum(g0[...][:, 0:1], 1.0)
    inv1 = 1.0 / jnp.maximum(g1[...][:, 0:1], 1.0)
    inv2 = 1.0 / jnp.maximum(g2[...][:, 0:1], 1.0)
    acc = jnp.dot(a0[...] * inv0, w0[...], preferred_element_type=jnp.float32)
    acc = acc + jnp.dot(a1[...] * inv1, w1[...], preferred_element_type=jnp.float32)
    acc = acc + jnp.dot(a2[...] * inv2, w2[...], preferred_element_type=jnp.float32)
    acc = acc + (b0[...] + b1[...] + b2[...])
    o[...] = jnp.maximum(acc, 0.0)


def _tc_layer(aggs, degs, ws, bs):
    row = lambda i: (i, 0)
    fix = lambda i: (0, 0)
    return pl.pallas_call(
        _tc_body,
        grid=(_GRID,),
        in_specs=[
            pl.BlockSpec((BLK, D), row), pl.BlockSpec((BLK, D), row),
            pl.BlockSpec((BLK, D), row),
            pl.BlockSpec((BLK, HW), row), pl.BlockSpec((BLK, HW), row),
            pl.BlockSpec((BLK, HW), row),
            pl.BlockSpec((D, D), fix), pl.BlockSpec((D, D), fix),
            pl.BlockSpec((D, D), fix),
            pl.BlockSpec((1, D), fix), pl.BlockSpec((1, D), fix),
            pl.BlockSpec((1, D), fix),
        ],
        out_specs=pl.BlockSpec((BLK, D), row),
        out_shape=jax.ShapeDtypeStruct((N, D), jnp.float32),
    )(*aggs, *degs, *ws, bs[0].reshape(1, D), bs[1].reshape(1, D),
      bs[2].reshape(1, D))


def kernel(x, edge_index_r0, edge_index_r1, edge_index_r2,
           W0_r0, b0_r0, W0_r1, b0_r1, W0_r2, b0_r2,
           W1_r0, b1_r0, W1_r1, b1_r1, W1_r2, b1_r2):
    s0, d0 = edge_index_r0[0], edge_index_r0[1]
    s1, d1 = edge_index_r1[0], edge_index_r1[1]
    s2, d2 = edge_index_r2[0], edge_index_r2[1]
    z = jnp.zeros((ZROWS, HW), jnp.float32)
    o = jnp.ones((K, HW), jnp.float32)

    outs = _sc_l0(x, s0, d0, s1, d1, s2, d2, z, o)
    aggs, degs = outs[0:3], outs[3:6]
    h = _tc_layer(aggs, degs, (W0_r0, W0_r1, W0_r2), (b0_r0, b0_r1, b0_r2))
    aggs = _sc_l1(h, s0, d0, s1, d1, s2, d2, z, o)
    return _tc_layer(aggs, degs, (W1_r0, W1_r1, W1_r2), (b1_r0, b1_r1, b1_r2))


# speedup vs baseline: 2.4887x; 1.5938x over previous
import functools

import jax
import jax.numpy as jnp
from jax import lax
from jax.experimental import pallas as pl
from jax.experimental.pallas import tpu as pltpu
from jax.experimental.pallas import tpu_sc as plsc

N = 10000
D = 256
HW = 128
E = 64000

NC = 2
NS = 16
HALF = N // NC
ACC_ROWS = 5120
GARBAGE = HALF
ZROWS = ACC_ROWS // NS
EPT = E // NS
K = 80
NCHUNK = EPT // K
OUT_T = 5
OUT_ROWS = HALF // OUT_T


def _sc_segsum(compute_deg):
    mesh = plsc.VectorSubcoreMesh(core_axis_name="c", subcore_axis_name="s")
    out_type = [jax.ShapeDtypeStruct((N, D), jnp.float32) for _ in range(3)]
    if compute_deg:
        out_type += [jax.ShapeDtypeStruct((N, HW), jnp.float32) for _ in range(3)]
    scratch = [
        pltpu.VMEM((NCHUNK, K), jnp.int32),
        pltpu.VMEM((NCHUNK, K), jnp.int32),
        pltpu.VMEM((NCHUNK, K), jnp.int32),
        pltpu.VMEM((K, HW), jnp.float32),
        pltpu.VMEM((K, HW), jnp.float32),
        pltpu.VMEM((K, HW), jnp.float32),
        pltpu.VMEM_SHARED((ACC_ROWS, HW), jnp.float32),
        pltpu.SemaphoreType.DMA,
        pltpu.SemaphoreType.DMA,
        pltpu.SemaphoreType.DMA,
        pltpu.SemaphoreType.DMA,
    ]

    @functools.partial(pl.kernel, mesh=mesh, out_type=out_type,
                       scratch_types=scratch)
    def k(feat_h, s0, d0, s1, d1, s2, d2, z_h, o_h, *rest):
        aggs = rest[0:3]
        degs = rest[3:6] if compute_deg else (None, None, None)
        (srcv, dstv, ldv, rA, rB, onesb,
         acc, gsemA, gsemB, ssemA, ssemB) = rest[6 if compute_deg else 3:]

        c = lax.axis_index("c")
        s = lax.axis_index("s")
        lo = c * HALF
        bufs = ((rA, gsemA, ssemA), (rB, gsemB, ssemB))
        dummy = feat_h.at[pl.ds(0, K), pl.ds(0, HW)]

        def routemk(i, carry):
            for g in range(K // 16):
                dvec = dstv[i, pl.ds(g * 16, 16)]
                m = (dvec >= lo) & (dvec < lo + HALF)
                ldv[i, pl.ds(g * 16, 16)] = jnp.where(m, dvec - lo, GARBAGE)
            return carry

        def zero_acc():
            pltpu.sync_copy(z_h, acc.at[pl.ds(s * ZROWS, ZROWS)])

        WIN = 8

        if compute_deg:
            pltpu.sync_copy(o_h, onesb)
            for (dsth, degh) in zip((d0, d1, d2), degs):
                pltpu.sync_copy(dsth.at[s], dstv)
                lax.fori_loop(0, NCHUNK, routemk, 0)
                zero_acc()
                plsc.subcore_barrier()

                def dbody(i, carry):
                    pltpu.async_copy(onesb, acc.at[ldv.at[i]], ssemA, add=True)

                    @pl.when(i >= WIN)
                    def _():
                        pltpu.make_async_copy(dummy, onesb, ssemA).wait()

                    return carry

                lax.fori_loop(0, NCHUNK, dbody, 0)

                def ddrain(i, carry):
                    pltpu.make_async_copy(dummy, onesb, ssemA).wait()
                    return carry

                lax.fori_loop(0, WIN, ddrain, 0)
                plsc.subcore_barrier()

                @pl.when(s < OUT_T)
                def _():
                    r0 = pl.multiple_of(s * OUT_ROWS, 8)
                    pltpu.sync_copy(acc.at[pl.ds(r0, OUT_ROWS)],
                                    degh.at[pl.ds(lo + r0, OUT_ROWS)])

                plsc.subcore_barrier()

        for (srch, dsth, aggh) in zip((s0, s1, s2), (d0, d1, d2), aggs):
            pltpu.sync_copy(srch.at[s], srcv)
            pltpu.sync_copy(dsth.at[s], dstv)
            lax.fori_loop(0, NCHUNK, routemk, 0)

            for cb in (0, HW):
                def issue_gather(i, buf):
                    r_, gsem, _ = buf
                    pltpu.async_copy(feat_h.at[srcv.at[i], pl.ds(cb, HW)],
                                     r_, gsem)

                def wait_gather(buf):
                    r_, gsem, _ = buf
                    pltpu.make_async_copy(dummy, r_, gsem).wait()

                def issue_scatter(i, buf):
                    r_, _, ssem = buf
                    pltpu.async_copy(r_, acc.at[ldv.at[i]], ssem, add=True)

                def wait_scatter(buf):
                    r_, _, ssem = buf
                    pltpu.make_async_copy(dummy, r_, ssem).wait()

                zero_acc()
                plsc.subcore_barrier()

                issue_gather(0, bufs[0])
                issue_gather(1, bufs[1])

                def body(j, carry):
                    for p in (0, 1):
                        i = 2 * j + p
                        wait_gather(bufs[p])
                        issue_scatter(i, bufs[p])

                        @pl.when(i + 2 < NCHUNK)
                        def _():
                            wait_scatter(bufs[p])
                            issue_gather(i + 2, bufs[p])

                    return carry

                lax.fori_loop(0, NCHUNK // 2, body, 0)
                wait_scatter(bufs[0])
                wait_scatter(bufs[1])
                plsc.subcore_barrier()

                @pl.when(s < OUT_T)
                def _():
                    r0 = pl.multiple_of(s * OUT_ROWS, 8)
                    pltpu.sync_copy(acc.at[pl.ds(r0, OUT_ROWS)],
                                    aggh.at[pl.ds(lo + r0, OUT_ROWS),
                                            pl.ds(cb, HW)])

                plsc.subcore_barrier()

    return k


_sc_l0 = _sc_segsum(compute_deg=True)
_sc_l1 = _sc_segsum(compute_deg=False)


BLK = 512
_GRID = (N + BLK - 1) // BLK


def _tc_body(a0, a1, a2, g0, g1, g2, w0, w1, w2, b0, b1, b2, o):
    inv0 = 1.0 / jnp.maximum(g0[...][:, 0:1], 1.0)
    inv1 = 1.0 / jnp.maximum(g1[...][:, 0:1], 1.0)
    inv2 = 1.0 / jnp.maximum(g2[...][:, 0:1], 1.0)
    acc = jnp.dot(a0[...] * inv0, w0[...], preferred_element_type=jnp.float32)
    acc = acc + jnp.dot(a1[...] * inv1, w1[...], preferred_element_type=jnp.float32)
    acc = acc + jnp.dot(a2[...] * inv2, w2[...], preferred_element_type=jnp.float32)
    acc = acc + (b0[...] + b1[...] + b2[...])
    o[...] = jnp.maximum(acc, 0.0)


def _tc_layer(aggs, degs, ws, bs):
    row = lambda i: (i, 0)
    fix = lambda i: (0, 0)
    return pl.pallas_call(
        _tc_body,
        grid=(_GRID,),
        in_specs=[
            pl.BlockSpec((BLK, D), row), pl.BlockSpec((BLK, D), row),
            pl.BlockSpec((BLK, D), row),
            pl.BlockSpec((BLK, HW), row), pl.BlockSpec((BLK, HW), row),
            pl.BlockSpec((BLK, HW), row),
            pl.BlockSpec((D, D), fix), pl.BlockSpec((D, D), fix),
            pl.BlockSpec((D, D), fix),
            pl.BlockSpec((1, D), fix), pl.BlockSpec((1, D), fix),
            pl.BlockSpec((1, D), fix),
        ],
        out_specs=pl.BlockSpec((BLK, D), row),
        out_shape=jax.ShapeDtypeStruct((N, D), jnp.float32),
    )(*aggs, *degs, *ws, bs[0].reshape(1, D), bs[1].reshape(1, D),
      bs[2].reshape(1, D))


def kernel(x, edge_index_r0, edge_index_r1, edge_index_r2,
           W0_r0, b0_r0, W0_r1, b0_r1, W0_r2, b0_r2,
           W1_r0, b1_r0, W1_r1, b1_r1, W1_r2, b1_r2):
    shp = (NS, NCHUNK, K)
    s0, d0 = edge_index_r0[0].reshape(shp), edge_index_r0[1].reshape(shp)
    s1, d1 = edge_index_r1[0].reshape(shp), edge_index_r1[1].reshape(shp)
    s2, d2 = edge_index_r2[0].reshape(shp), edge_index_r2[1].reshape(shp)
    z = jnp.zeros((ZROWS, HW), jnp.float32)
    o = jnp.ones((K, HW), jnp.float32)

    outs = _sc_l0(x, s0, d0, s1, d1, s2, d2, z, o)
    aggs, degs = outs[0:3], outs[3:6]
    h = _tc_layer(aggs, degs, (W0_r0, W0_r1, W0_r2), (b0_r0, b0_r1, b0_r2))
    aggs = _sc_l1(h, s0, d0, s1, d1, s2, d2, z, o)
    return _tc_layer(aggs, degs, (W1_r0, W1_r1, W1_r2), (b1_r0, b1_r1, b1_r2))
